# baseline (speedup 1.0000x reference)
import jax
import jax.numpy as jnp
from jax import lax
from jax.experimental import pallas as pl
from jax.experimental.pallas import tpu as pltpu
from jax.experimental.pallas import tpu_sc as plsc

_B, _S, _D = 4, 4096, 1024
_NC, _NS = 2, 16
_NW = _NC * _NS
_RPW = _S // _NW
_CHR = 16
_CH = _CHR * _D
_NCHUNK = _RPW // _CHR
_NSTEP = _NCHUNK * _B
_NBUF = 4


def _sc_body(x_hbm, pos_hbm, out_hbm, b0, b1, b2, b3, pos0, pos1, *sems):
    bufs = (b0, b1, b2, b3)
    pos_buf = (pos0, pos1)
    xsem, osem, psem = sems[0:4], sems[4:8], sems[8:10]
    wid = lax.axis_index("s") * _NC + lax.axis_index("c")
    base = wid * _RPW * _D

    def x_off(step):
        return (step % _B) * _S * _D + base + (step // _B) * _CH

    def issue_in(step):
        r = step % _NBUF
        return pltpu.async_copy(x_hbm.at[pl.ds(x_off(step), _CH)],
                                bufs[r], xsem[r])

    def issue_out(step):
        r = step % _NBUF
        return pltpu.async_copy(bufs[r], out_hbm.at[pl.ds(x_off(step), _CH)],
                                osem[r])

    def issue_pos(ci):
        return pltpu.async_copy(pos_hbm.at[pl.ds(base + ci * _CH, _CH)],
                                pos_buf[ci % 2], psem[ci % 2])

    pos_dma = [None] * _NCHUNK
    in_dma = [None] * _NSTEP
    out_dma = [None] * _NSTEP
    pos_dma[0] = issue_pos(0)
    in_dma[0] = issue_in(0)
    in_dma[1] = issue_in(1)

    for s in range(_NSTEP):
        ci, b = s // _B, s % _B
        r, pc = s % _NBUF, ci % 2
        if b == 0:
            pos_dma[ci].wait()
            if ci + 1 < _NCHUNK:
                pos_dma[ci + 1] = issue_pos(ci + 1)
        in_dma[s].wait()
        buf, pbuf = bufs[r], pos_buf[pc]

        @plsc.parallel_loop(0, _CH, 16, unroll=16)
        def _add(i):
            plsc.addupdate(buf.at[pl.ds(i, 16)], pbuf[pl.ds(i, 16)])

        out_dma[s] = issue_out(s)
        if s + 2 - _NBUF >= 0:
            out_dma[s + 2 - _NBUF].wait()
        if s + 2 < _NSTEP:
            in_dma[s + 2] = issue_in(s + 2)

    out_dma[_NSTEP - 2].wait()
    out_dma[_NSTEP - 1].wait()


def kernel(x, pos_table):
    mesh = plsc.VectorSubcoreMesh(core_axis_name="c", subcore_axis_name="s")
    k = pl.kernel(
        _sc_body,
        out_type=jax.ShapeDtypeStruct((_B * _S * _D,), jnp.float32),
        mesh=mesh,
        scratch_types=(
            [pltpu.VMEM((_CH,), jnp.float32)] * (_NBUF + 2)
            + [pltpu.SemaphoreType.DMA] * 10
        ),
    )
    out = k(x.reshape(-1), pos_table.reshape(-1))
    return out.reshape(x.shape)


# speedup vs baseline: 2.7277x; 2.7277x over previous
import jax
import jax.numpy as jnp
from jax import lax
from jax.experimental import pallas as pl
from jax.experimental.pallas import tpu as pltpu
from jax.experimental.pallas import tpu_sc as plsc

_B, _S, _D = 4, 4096, 1024
_NC, _NS = 2, 16
_NW = _NC * _NS
_RPW = _S // _NW
_CHR = 16
_NCHUNK = _RPW // _CHR
_NSTEP = _NCHUNK * _B
_NBUF = 4


def _sc_body(x_hbm, pos_hbm, out_hbm, b0, b1, b2, b3, pos0, pos1, *sems):
    bufs = (b0, b1, b2, b3)
    pos_buf = (pos0, pos1)
    xsem, osem, psem = sems[0:4], sems[4:8], sems[8:10]
    wid = lax.axis_index("s") * _NC + lax.axis_index("c")
    row0 = wid * _RPW

    def rows(step):
        return pl.ds(row0 + (step // _B) * _CHR, _CHR)

    def issue_in(step):
        r = step % _NBUF
        return pltpu.async_copy(x_hbm.at[step % _B, rows(step), :],
                                bufs[r], xsem[r])

    def issue_out(step):
        r = step % _NBUF
        return pltpu.async_copy(bufs[r], out_hbm.at[step % _B, rows(step), :],
                                osem[r])

    def issue_pos(ci):
        return pltpu.async_copy(
            pos_hbm.at[pl.ds(row0 + ci * _CHR, _CHR), :],
            pos_buf[ci % 2], psem[ci % 2])

    pos_dma = [None] * _NCHUNK
    in_dma = [None] * _NSTEP
    out_dma = [None] * _NSTEP
    pos_dma[0] = issue_pos(0)
    in_dma[0] = issue_in(0)
    in_dma[1] = issue_in(1)

    for s in range(_NSTEP):
        ci, b = s // _B, s % _B
        r, pc = s % _NBUF, ci % 2
        if b == 0:
            pos_dma[ci].wait()
            if ci + 1 < _NCHUNK:
                pos_dma[ci + 1] = issue_pos(ci + 1)
        in_dma[s].wait()
        buf, pbuf = bufs[r], pos_buf[pc]

        @plsc.parallel_loop(0, _CHR * _D, 16, unroll=8)
        def _add(i):
            rr = jax.lax.shift_right_logical(i, 10)
            cc = pl.multiple_of(jax.lax.bitwise_and(i, _D - 1), 16)
            plsc.addupdate(buf.at[rr, pl.ds(cc, 16)],
                           pbuf[rr, pl.ds(cc, 16)])

        out_dma[s] = issue_out(s)
        if s + 2 - _NBUF >= 0:
            out_dma[s + 2 - _NBUF].wait()
        if s + 2 < _NSTEP:
            in_dma[s + 2] = issue_in(s + 2)

    out_dma[_NSTEP - 2].wait()
    out_dma[_NSTEP - 1].wait()


def kernel(x, pos_table):
    mesh = plsc.VectorSubcoreMesh(core_axis_name="c", subcore_axis_name="s")
    k = pl.kernel(
        _sc_body,
        out_type=jax.ShapeDtypeStruct((_B, _S, _D), jnp.float32),
        mesh=mesh,
        scratch_types=(
            [pltpu.VMEM((_CHR, _D), jnp.float32)] * (_NBUF + 2)
            + [pltpu.SemaphoreType.DMA] * 10
        ),
    )
    return k(x, pos_table)
